# Initial kernel scaffold; baseline (speedup 1.0000x reference)
import functools

import jax
import jax.numpy as jnp
from jax import lax
from jax.experimental import pallas as pl
from jax.experimental.pallas import tpu as pltpu
from jax.experimental.pallas import tpu_sc as plsc

N = 10000
E = 320000
D = 128
NC = 2
NS = 16
NW = NC * NS
EW = E // NW
C = 80
STRIPE = N // NS

_mesh = plsc.VectorSubcoreMesh(core_axis_name="c", subcore_axis_name="s")


@functools.partial(
    pl.kernel,
    out_type=jax.ShapeDtypeStruct((E, D), jnp.float32),
    mesh=_mesh,
    scratch_types=[
        pltpu.VMEM((C,), jnp.int32),
        pltpu.VMEM((C,), jnp.int32),
        pltpu.VMEM((C, D), jnp.float32),
    ],
)
def _gather_sum(ps_hbm, pr_hbm, snd_hbm, rcv_hbm, out_hbm, idx_s, idx_r, rows):
    w = lax.axis_index("s") * NC + lax.axis_index("c")
    base0 = w * EW

    def body(i, carry):
        base = base0 + i * C
        pltpu.sync_copy(snd_hbm.at[pl.ds(base, C)], idx_s)
        pltpu.sync_copy(rcv_hbm.at[pl.ds(base, C)], idx_r)
        pltpu.sync_copy(ps_hbm.at[idx_s], rows)
        pltpu.sync_copy(pr_hbm.at[idx_r], rows, add=True)
        pltpu.sync_copy(rows, out_hbm.at[pl.ds(base, C), :])
        return carry

    lax.fori_loop(0, EW // C, body, 0)


@functools.partial(
    pl.kernel,
    out_type=jax.ShapeDtypeStruct((NC * N, D), jnp.float32),
    mesh=_mesh,
    scratch_types=[
        pltpu.VMEM((C,), jnp.int32),
        pltpu.VMEM((C, D), jnp.float32),
        pltpu.VMEM_SHARED((N, D), jnp.float32),
    ],
)
def _seg_sum(edge_hbm, rcv_hbm, zeros_hbm, out_hbm, idx, rows, acc):
    c = lax.axis_index("c")
    s = lax.axis_index("s")
    w = s * NC + c
    base0 = w * EW
    pltpu.sync_copy(zeros_hbm, acc.at[pl.ds(s * STRIPE, STRIPE), :])
    plsc.subcore_barrier()

    def body(i, carry):
        base = base0 + i * C
        pltpu.sync_copy(rcv_hbm.at[pl.ds(base, C)], idx)
        pltpu.sync_copy(edge_hbm.at[pl.ds(base, C), :], rows)
        pltpu.sync_copy(rows, acc.at[idx], add=True)
        return carry

    lax.fori_loop(0, EW // C, body, 0)
    plsc.subcore_barrier()
    pltpu.sync_copy(
        acc.at[pl.ds(s * STRIPE, STRIPE), :],
        out_hbm.at[pl.ds(c * N + s * STRIPE, STRIPE), :],
    )


BN = 1000
BEDGE = 1000


def _proj_body(nf, w1s, w1r, ps, pr):
    x = nf[...]
    ps[...] = jnp.dot(x, w1s[...], preferred_element_type=jnp.float32)
    pr[...] = jnp.dot(x, w1r[...], preferred_element_type=jnp.float32)


_proj = pl.pallas_call(
    _proj_body,
    grid=(N // BN,),
    in_specs=[
        pl.BlockSpec((BN, D), lambda i: (i, 0)),
        pl.BlockSpec((D, D), lambda i: (0, 0)),
        pl.BlockSpec((D, D), lambda i: (0, 0)),
    ],
    out_specs=[pl.BlockSpec((BN, D), lambda i: (i, 0))] * 2,
    out_shape=[jax.ShapeDtypeStruct((N, D), jnp.float32)] * 2,
)


def _layer_norm(h, g, beta):
    mu = jnp.mean(h, axis=-1, keepdims=True)
    d = h - mu
    var = jnp.mean(d * d, axis=-1, keepdims=True)
    return d * lax.rsqrt(var + 1e-5) * g + beta


def _edge_mlp_body(gath, ef, w1e, b1, w2, b2, g, beta, new_edge, out_edge):
    e = ef[...]
    h = gath[...] + jnp.dot(e, w1e[...], preferred_element_type=jnp.float32)
    h = jnp.maximum(h + b1[...], 0.0)
    h = jnp.dot(h, w2[...], preferred_element_type=jnp.float32) + b2[...]
    y = _layer_norm(h, g[...], beta[...])
    new_edge[...] = y
    out_edge[...] = y + e


_edge_mlp = pl.pallas_call(
    _edge_mlp_body,
    grid=(E // BEDGE,),
    in_specs=[
        pl.BlockSpec((BEDGE, D), lambda i: (i, 0)),
        pl.BlockSpec((BEDGE, D), lambda i: (i, 0)),
        pl.BlockSpec((D, D), lambda i: (0, 0)),
        pl.BlockSpec((1, D), lambda i: (0, 0)),
        pl.BlockSpec((D, D), lambda i: (0, 0)),
        pl.BlockSpec((1, D), lambda i: (0, 0)),
        pl.BlockSpec((1, D), lambda i: (0, 0)),
        pl.BlockSpec((1, D), lambda i: (0, 0)),
    ],
    out_specs=[pl.BlockSpec((BEDGE, D), lambda i: (i, 0))] * 2,
    out_shape=[jax.ShapeDtypeStruct((E, D), jnp.float32)] * 2,
)


def _node_mlp_body(nf, p0, p1, w1a, w1b, b1, w2, b2, g, beta, out):
    x = nf[...]
    seg = p0[...] + p1[...]
    h = jnp.dot(x, w1a[...], preferred_element_type=jnp.float32)
    h = h + jnp.dot(seg, w1b[...], preferred_element_type=jnp.float32)
    h = jnp.maximum(h + b1[...], 0.0)
    h = jnp.dot(h, w2[...], preferred_element_type=jnp.float32) + b2[...]
    out[...] = _layer_norm(h, g[...], beta[...]) + x


_node_mlp = pl.pallas_call(
    _node_mlp_body,
    grid=(N // BN,),
    in_specs=[
        pl.BlockSpec((BN, D), lambda i: (i, 0)),
        pl.BlockSpec((BN, D), lambda i: (i, 0)),
        pl.BlockSpec((BN, D), lambda i: (i, 0)),
        pl.BlockSpec((D, D), lambda i: (0, 0)),
        pl.BlockSpec((D, D), lambda i: (0, 0)),
        pl.BlockSpec((1, D), lambda i: (0, 0)),
        pl.BlockSpec((D, D), lambda i: (0, 0)),
        pl.BlockSpec((1, D), lambda i: (0, 0)),
        pl.BlockSpec((1, D), lambda i: (0, 0)),
        pl.BlockSpec((1, D), lambda i: (0, 0)),
    ],
    out_specs=pl.BlockSpec((BN, D), lambda i: (i, 0)),
    out_shape=jax.ShapeDtypeStruct((N, D), jnp.float32),
)


def kernel(node_features, edge_features, senders, receivers,
           edge_w1, edge_b1, edge_w2, edge_b2, edge_g, edge_beta,
           node_w1, node_b1, node_w2, node_b2, node_g, node_beta):
    ps, pr = _proj(node_features, edge_w1[:D], edge_w1[D:2 * D])
    gath = _gather_sum(ps, pr, senders, receivers)
    new_edge, out_edge = _edge_mlp(
        gath, edge_features, edge_w1[2 * D:], edge_b1.reshape(1, D),
        edge_w2, edge_b2.reshape(1, D), edge_g.reshape(1, D),
        edge_beta.reshape(1, D))
    zeros = jnp.zeros((STRIPE, D), jnp.float32)
    parts = _seg_sum(new_edge, receivers, zeros)
    out_node = _node_mlp(
        node_features, parts[:N], parts[N:], node_w1[:D], node_w1[D:],
        node_b1.reshape(1, D), node_w2, node_b2.reshape(1, D),
        node_g.reshape(1, D), node_beta.reshape(1, D))
    return (out_node, out_edge)


# speedup vs baseline: 2.7527x; 2.7527x over previous
import functools

import jax
import jax.numpy as jnp
from jax import lax
from jax.experimental import pallas as pl
from jax.experimental.pallas import tpu as pltpu
from jax.experimental.pallas import tpu_sc as plsc

N = 10000
E = 320000
D = 128
NC = 2
NS = 16
NW = NC * NS
EW = E // NW
C = 80
STRIPE = 632
NP = NS * STRIPE

_mesh = plsc.VectorSubcoreMesh(core_axis_name="c", subcore_axis_name="s")


@functools.partial(
    pl.kernel,
    out_type=jax.ShapeDtypeStruct((E, D), jnp.float32),
    mesh=_mesh,
    scratch_types=[
        pltpu.VMEM((C,), jnp.int32),
        pltpu.VMEM((C,), jnp.int32),
        pltpu.VMEM((C, D), jnp.float32),
    ],
)
def _gather_sum(ps_hbm, pr_hbm, snd_hbm, rcv_hbm, out_hbm, idx_s, idx_r, rows):
    w = lax.axis_index("s") * NC + lax.axis_index("c")
    base0 = w * EW

    def body(i, carry):
        base = base0 + i * C
        pltpu.sync_copy(snd_hbm.at[pl.ds(base, C)], idx_s)
        pltpu.sync_copy(rcv_hbm.at[pl.ds(base, C)], idx_r)
        pltpu.sync_copy(ps_hbm.at[idx_s], rows)
        pltpu.sync_copy(pr_hbm.at[idx_r], rows, add=True)
        pltpu.sync_copy(rows, out_hbm.at[pl.ds(base, C), :])
        return carry

    lax.fori_loop(0, EW // C, body, 0)


@functools.partial(
    pl.kernel,
    out_type=jax.ShapeDtypeStruct((NC * NP, D), jnp.float32),
    mesh=_mesh,
    scratch_types=[
        pltpu.VMEM((C,), jnp.int32),
        pltpu.VMEM((C, D), jnp.float32),
        pltpu.VMEM_SHARED((NP, D), jnp.float32),
    ],
)
def _seg_sum(edge_hbm, rcv_hbm, zeros_hbm, out_hbm, idx, rows, acc):
    c = lax.axis_index("c")
    s = lax.axis_index("s")
    w = s * NC + c
    base0 = w * EW
    pltpu.sync_copy(zeros_hbm, acc.at[pl.ds(s * STRIPE, STRIPE), :])
    plsc.subcore_barrier()

    def body(i, carry):
        base = base0 + i * C
        pltpu.sync_copy(rcv_hbm.at[pl.ds(base, C)], idx)
        pltpu.sync_copy(edge_hbm.at[pl.ds(base, C), :], rows)
        pltpu.sync_copy(rows, acc.at[idx], add=True)
        return carry

    lax.fori_loop(0, EW // C, body, 0)
    plsc.subcore_barrier()
    pltpu.sync_copy(
        acc.at[pl.ds(s * STRIPE, STRIPE), :],
        out_hbm.at[pl.ds(c * NP + s * STRIPE, STRIPE), :],
    )


BN = 1000
BEDGE = 1000


def _proj_body(nf, w1s, w1r, ps, pr):
    x = nf[...]
    ps[...] = jnp.dot(x, w1s[...], preferred_element_type=jnp.float32)
    pr[...] = jnp.dot(x, w1r[...], preferred_element_type=jnp.float32)


_proj = pl.pallas_call(
    _proj_body,
    grid=(N // BN,),
    in_specs=[
        pl.BlockSpec((BN, D), lambda i: (i, 0)),
        pl.BlockSpec((D, D), lambda i: (0, 0)),
        pl.BlockSpec((D, D), lambda i: (0, 0)),
    ],
    out_specs=[pl.BlockSpec((BN, D), lambda i: (i, 0))] * 2,
    out_shape=[jax.ShapeDtypeStruct((N, D), jnp.float32)] * 2,
)


def _layer_norm(h, g, beta):
    mu = jnp.mean(h, axis=-1, keepdims=True)
    d = h - mu
    var = jnp.mean(d * d, axis=-1, keepdims=True)
    return d * lax.rsqrt(var + 1e-5) * g + beta


def _edge_mlp_body(gath, ef, w1e, b1, w2, b2, g, beta, new_edge, out_edge):
    e = ef[...]
    h = gath[...] + jnp.dot(e, w1e[...], preferred_element_type=jnp.float32)
    h = jnp.maximum(h + b1[...], 0.0)
    h = jnp.dot(h, w2[...], preferred_element_type=jnp.float32) + b2[...]
    y = _layer_norm(h, g[...], beta[...])
    new_edge[...] = y
    out_edge[...] = y + e


_edge_mlp = pl.pallas_call(
    _edge_mlp_body,
    grid=(E // BEDGE,),
    in_specs=[
        pl.BlockSpec((BEDGE, D), lambda i: (i, 0)),
        pl.BlockSpec((BEDGE, D), lambda i: (i, 0)),
        pl.BlockSpec((D, D), lambda i: (0, 0)),
        pl.BlockSpec((1, D), lambda i: (0, 0)),
        pl.BlockSpec((D, D), lambda i: (0, 0)),
        pl.BlockSpec((1, D), lambda i: (0, 0)),
        pl.BlockSpec((1, D), lambda i: (0, 0)),
        pl.BlockSpec((1, D), lambda i: (0, 0)),
    ],
    out_specs=[pl.BlockSpec((BEDGE, D), lambda i: (i, 0))] * 2,
    out_shape=[jax.ShapeDtypeStruct((E, D), jnp.float32)] * 2,
)


def _node_mlp_body(nf, p0, p1, w1a, w1b, b1, w2, b2, g, beta, out):
    x = nf[...]
    seg = p0[...] + p1[...]
    h = jnp.dot(x, w1a[...], preferred_element_type=jnp.float32)
    h = h + jnp.dot(seg, w1b[...], preferred_element_type=jnp.float32)
    h = jnp.maximum(h + b1[...], 0.0)
    h = jnp.dot(h, w2[...], preferred_element_type=jnp.float32) + b2[...]
    out[...] = _layer_norm(h, g[...], beta[...]) + x


_node_mlp = pl.pallas_call(
    _node_mlp_body,
    grid=(N // BN,),
    in_specs=[
        pl.BlockSpec((BN, D), lambda i: (i, 0)),
        pl.BlockSpec((BN, D), lambda i: (i, 0)),
        pl.BlockSpec((BN, D), lambda i: (i, 0)),
        pl.BlockSpec((D, D), lambda i: (0, 0)),
        pl.BlockSpec((D, D), lambda i: (0, 0)),
        pl.BlockSpec((1, D), lambda i: (0, 0)),
        pl.BlockSpec((D, D), lambda i: (0, 0)),
        pl.BlockSpec((1, D), lambda i: (0, 0)),
        pl.BlockSpec((1, D), lambda i: (0, 0)),
        pl.BlockSpec((1, D), lambda i: (0, 0)),
    ],
    out_specs=pl.BlockSpec((BN, D), lambda i: (i, 0)),
    out_shape=jax.ShapeDtypeStruct((N, D), jnp.float32),
)


def kernel(node_features, edge_features, senders, receivers,
           edge_w1, edge_b1, edge_w2, edge_b2, edge_g, edge_beta,
           node_w1, node_b1, node_w2, node_b2, node_g, node_beta):
    ps, pr = _proj(node_features, edge_w1[:D], edge_w1[D:2 * D])
    gath = _gather_sum(ps, pr, senders, receivers)
    new_edge, out_edge = _edge_mlp(
        gath, edge_features, edge_w1[2 * D:], edge_b1.reshape(1, D),
        edge_w2, edge_b2.reshape(1, D), edge_g.reshape(1, D),
        edge_beta.reshape(1, D))
    zeros = jnp.zeros((STRIPE, D), jnp.float32)
    parts = _seg_sum(new_edge, receivers, zeros)
    out_node = _node_mlp(
        node_features, parts[:N], parts[NP:NP + N], node_w1[:D], node_w1[D:],
        node_b1.reshape(1, D), node_w2, node_b2.reshape(1, D),
        node_g.reshape(1, D), node_beta.reshape(1, D))
    return (out_node, out_edge)


# speedup vs baseline: 3.6290x; 1.3183x over previous
import functools

import jax
import jax.numpy as jnp
from jax import lax
from jax.experimental import pallas as pl
from jax.experimental.pallas import tpu as pltpu
from jax.experimental.pallas import tpu_sc as plsc

N = 10000
E = 320000
D = 128
NC = 2
NS = 16
NW = NC * NS
EW = E // NW
C = 80
NCH = EW // C
CS = 40
NCHS = EW // CS
NBUF = 5
STRIPE = 632
NP = NS * STRIPE

_mesh = plsc.VectorSubcoreMesh(core_axis_name="c", subcore_axis_name="s")


@functools.partial(
    pl.kernel,
    out_type=jax.ShapeDtypeStruct((E, D), jnp.float32),
    mesh=_mesh,
    scratch_types=[
        pltpu.VMEM((NCH, C), jnp.int32),
        pltpu.VMEM((NCH, C), jnp.int32),
        pltpu.VMEM((NBUF, C, D), jnp.float32),
        pltpu.SemaphoreType.DMA((NBUF,)),
        pltpu.SemaphoreType.DMA((NBUF,)),
        pltpu.SemaphoreType.DMA((NBUF,)),
    ],
)
def _gather_sum(ps_hbm, pr_hbm, snd_hbm, rcv_hbm, out_hbm,
                idx_s, idx_r, bufs, gsem, asem, wsem):
    w = lax.axis_index("s") * NC + lax.axis_index("c")
    base0 = w * EW
    pltpu.sync_copy(snd_hbm.at[w], idx_s)
    pltpu.sync_copy(rcv_hbm.at[w], idx_r)

    def gs(j, b):
        pltpu.async_copy(ps_hbm.at[idx_s.at[j]], bufs.at[b], gsem.at[b])

    def ga(j, b):
        pltpu.make_async_copy(ps_hbm.at[idx_s.at[j]], bufs.at[b],
                              gsem.at[b]).wait()
        pltpu.async_copy(pr_hbm.at[idx_r.at[j]], bufs.at[b], asem.at[b],
                         add=True)

    def wr(j, b):
        pltpu.make_async_copy(pr_hbm.at[idx_r.at[j]], bufs.at[b],
                              asem.at[b]).wait()
        pltpu.async_copy(bufs.at[b], out_hbm.at[pl.ds(base0 + j * C, C), :],
                         wsem.at[b])

    def wr_wait(j, b):
        pltpu.make_async_copy(bufs.at[b],
                              out_hbm.at[pl.ds(base0 + j * C, C), :],
                              wsem.at[b]).wait()

    def outer(i, carry):
        for b in range(NBUF):
            j = i * NBUF + b

            @pl.when(j >= 2)
            def _():
                wr(j - 2, (b - 2) % NBUF)

            @pl.when(j >= 1)
            def _():
                ga(j - 1, (b - 1) % NBUF)

            @pl.when(j >= NBUF)
            def _():
                wr_wait(j - NBUF, b)

            gs(j, b)
        return carry

    lax.fori_loop(0, NCH // NBUF, outer, 0)
    ga(NCH - 1, (NCH - 1) % NBUF)
    wr(NCH - 2, (NCH - 2) % NBUF)
    wr(NCH - 1, (NCH - 1) % NBUF)
    for b in range(NBUF):
        wr_wait(NCH - NBUF + b, b)


@functools.partial(
    pl.kernel,
    out_type=jax.ShapeDtypeStruct((NC * NP, D), jnp.float32),
    mesh=_mesh,
    scratch_types=[
        pltpu.VMEM((NBUF, CS), jnp.int32),
        pltpu.VMEM((NBUF, CS, D), jnp.float32),
        pltpu.VMEM_SHARED((NP, D), jnp.float32),
        pltpu.SemaphoreType.DMA((NBUF,)),
        pltpu.SemaphoreType.DMA((NBUF,)),
        pltpu.SemaphoreType.DMA((NBUF,)),
    ],
)
def _seg_sum(edge_hbm, rcv_hbm, zeros_hbm, out_hbm, idx, bufs, acc,
             isem, lsem, ssem):
    c = lax.axis_index("c")
    s = lax.axis_index("s")
    w = s * NC + c
    base0 = w * EW
    pltpu.sync_copy(zeros_hbm, acc.at[pl.ds(s * STRIPE, STRIPE), :])
    plsc.subcore_barrier()

    def ld(j, b):
        pltpu.async_copy(rcv_hbm.at[pl.ds(base0 + j * CS, CS)], idx.at[b],
                         isem.at[b])
        pltpu.async_copy(edge_hbm.at[pl.ds(base0 + j * CS, CS), :], bufs.at[b],
                         lsem.at[b])

    def sadd(j, b):
        pltpu.make_async_copy(rcv_hbm.at[pl.ds(base0 + j * CS, CS)],
                              idx.at[b], isem.at[b]).wait()
        pltpu.make_async_copy(edge_hbm.at[pl.ds(base0 + j * CS, CS), :],
                              bufs.at[b], lsem.at[b]).wait()
        pltpu.async_copy(bufs.at[b], acc.at[idx.at[b]], ssem.at[b], add=True)

    def sadd_wait(b):
        pltpu.make_async_copy(bufs.at[b], acc.at[idx.at[b]],
                              ssem.at[b]).wait()

    def outer(i, carry):
        for b in range(NBUF):
            j = i * NBUF + b

            @pl.when(j >= 1)
            def _():
                sadd(j - 1, (b - 1) % NBUF)

            @pl.when(j >= NBUF)
            def _():
                sadd_wait(b)

            ld(j, b)
        return carry

    lax.fori_loop(0, NCHS // NBUF, outer, 0)
    sadd(NCHS - 1, (NCHS - 1) % NBUF)
    for b in range(NBUF):
        sadd_wait(b)
    plsc.subcore_barrier()
    pltpu.sync_copy(
        acc.at[pl.ds(s * STRIPE, STRIPE), :],
        out_hbm.at[pl.ds(c * NP + s * STRIPE, STRIPE), :],
    )


BN = 1000
BEDGE = 1000


def _proj_body(nf, w1s, w1r, ps, pr):
    x = nf[...]
    ps[...] = jnp.dot(x, w1s[...], preferred_element_type=jnp.float32)
    pr[...] = jnp.dot(x, w1r[...], preferred_element_type=jnp.float32)


_proj = pl.pallas_call(
    _proj_body,
    grid=(N // BN,),
    in_specs=[
        pl.BlockSpec((BN, D), lambda i: (i, 0)),
        pl.BlockSpec((D, D), lambda i: (0, 0)),
        pl.BlockSpec((D, D), lambda i: (0, 0)),
    ],
    out_specs=[pl.BlockSpec((BN, D), lambda i: (i, 0))] * 2,
    out_shape=[jax.ShapeDtypeStruct((N, D), jnp.float32)] * 2,
)


def _layer_norm(h, g, beta):
    mu = jnp.mean(h, axis=-1, keepdims=True)
    d = h - mu
    var = jnp.mean(d * d, axis=-1, keepdims=True)
    return d * lax.rsqrt(var + 1e-5) * g + beta


def _edge_mlp_body(gath, ef, w1e, b1, w2, b2, g, beta, new_edge, out_edge):
    e = ef[...]
    h = gath[...] + jnp.dot(e, w1e[...], preferred_element_type=jnp.float32)
    h = jnp.maximum(h + b1[...], 0.0)
    h = jnp.dot(h, w2[...], preferred_element_type=jnp.float32) + b2[...]
    y = _layer_norm(h, g[...], beta[...])
    new_edge[...] = y
    out_edge[...] = y + e


_edge_mlp = pl.pallas_call(
    _edge_mlp_body,
    grid=(E // BEDGE,),
    in_specs=[
        pl.BlockSpec((BEDGE, D), lambda i: (i, 0)),
        pl.BlockSpec((BEDGE, D), lambda i: (i, 0)),
        pl.BlockSpec((D, D), lambda i: (0, 0)),
        pl.BlockSpec((1, D), lambda i: (0, 0)),
        pl.BlockSpec((D, D), lambda i: (0, 0)),
        pl.BlockSpec((1, D), lambda i: (0, 0)),
        pl.BlockSpec((1, D), lambda i: (0, 0)),
        pl.BlockSpec((1, D), lambda i: (0, 0)),
    ],
    out_specs=[pl.BlockSpec((BEDGE, D), lambda i: (i, 0))] * 2,
    out_shape=[jax.ShapeDtypeStruct((E, D), jnp.float32)] * 2,
)


def _node_mlp_body(nf, p0, p1, w1a, w1b, b1, w2, b2, g, beta, out):
    x = nf[...]
    seg = p0[...] + p1[...]
    h = jnp.dot(x, w1a[...], preferred_element_type=jnp.float32)
    h = h + jnp.dot(seg, w1b[...], preferred_element_type=jnp.float32)
    h = jnp.maximum(h + b1[...], 0.0)
    h = jnp.dot(h, w2[...], preferred_element_type=jnp.float32) + b2[...]
    out[...] = _layer_norm(h, g[...], beta[...]) + x


_node_mlp = pl.pallas_call(
    _node_mlp_body,
    grid=(N // BN,),
    in_specs=[
        pl.BlockSpec((BN, D), lambda i: (i, 0)),
        pl.BlockSpec((BN, D), lambda i: (i, 0)),
        pl.BlockSpec((BN, D), lambda i: (i, 0)),
        pl.BlockSpec((D, D), lambda i: (0, 0)),
        pl.BlockSpec((D, D), lambda i: (0, 0)),
        pl.BlockSpec((1, D), lambda i: (0, 0)),
        pl.BlockSpec((D, D), lambda i: (0, 0)),
        pl.BlockSpec((1, D), lambda i: (0, 0)),
        pl.BlockSpec((1, D), lambda i: (0, 0)),
        pl.BlockSpec((1, D), lambda i: (0, 0)),
    ],
    out_specs=pl.BlockSpec((BN, D), lambda i: (i, 0)),
    out_shape=jax.ShapeDtypeStruct((N, D), jnp.float32),
)


def kernel(node_features, edge_features, senders, receivers,
           edge_w1, edge_b1, edge_w2, edge_b2, edge_g, edge_beta,
           node_w1, node_b1, node_w2, node_b2, node_g, node_beta):
    ps, pr = _proj(node_features, edge_w1[:D], edge_w1[D:2 * D])
    snd3 = senders.reshape(NW, NCH, C)
    rcv3 = receivers.reshape(NW, NCH, C)
    gath = _gather_sum(ps, pr, snd3, rcv3)
    new_edge, out_edge = _edge_mlp(
        gath, edge_features, edge_w1[2 * D:], edge_b1.reshape(1, D),
        edge_w2, edge_b2.reshape(1, D), edge_g.reshape(1, D),
        edge_beta.reshape(1, D))
    zeros = jnp.zeros((STRIPE, D), jnp.float32)
    parts = _seg_sum(new_edge, receivers, zeros)
    out_node = _node_mlp(
        node_features, parts[:N], parts[NP:NP + N], node_w1[:D], node_w1[D:],
        node_b1.reshape(1, D), node_w2, node_b2.reshape(1, D),
        node_g.reshape(1, D), node_beta.reshape(1, D))
    return (out_node, out_edge)


# speedup vs baseline: 4.1410x; 1.1411x over previous
import functools

import jax
import jax.numpy as jnp
from jax import lax
from jax.experimental import pallas as pl
from jax.experimental.pallas import tpu as pltpu
from jax.experimental.pallas import tpu_sc as plsc

N = 10000
E = 320000
D = 128
NC = 2
NS = 16
NW = NC * NS
NHALF = 2
EH = E // NHALF
EWH = EH // NW
CG = 40
NCH = EWH // CG
NBUF = 5
STRIPE = 632
NP = NS * STRIPE

_mesh = plsc.VectorSubcoreMesh(core_axis_name="c", subcore_axis_name="s")


@functools.partial(
    pl.kernel,
    out_type=jax.ShapeDtypeStruct((EH, D), jnp.float32),
    mesh=_mesh,
    scratch_types=[
        pltpu.VMEM((NCH, CG), jnp.int32),
        pltpu.VMEM((NCH, CG), jnp.int32),
        pltpu.VMEM((NBUF, CG, D), jnp.float32),
        pltpu.SemaphoreType.DMA((NBUF,)),
        pltpu.SemaphoreType.DMA((NBUF,)),
        pltpu.SemaphoreType.DMA((NBUF,)),
    ],
)
def _gather_sum(ps_hbm, pr_hbm, snd_hbm, rcv_hbm, out_hbm,
                idx_s, idx_r, bufs, gsem, asem, wsem):
    w = lax.axis_index("s") * NC + lax.axis_index("c")
    base0 = w * EWH
    pltpu.sync_copy(snd_hbm.at[w], idx_s)
    pltpu.sync_copy(rcv_hbm.at[w], idx_r)

    def gs(j, b):
        pltpu.async_copy(ps_hbm.at[idx_s.at[j]], bufs.at[b], gsem.at[b])

    def ga(j, b):
        pltpu.make_async_copy(ps_hbm.at[idx_s.at[j]], bufs.at[b],
                              gsem.at[b]).wait()
        pltpu.async_copy(pr_hbm.at[idx_r.at[j]], bufs.at[b], asem.at[b],
                         add=True)

    def wr(j, b):
        pltpu.make_async_copy(pr_hbm.at[idx_r.at[j]], bufs.at[b],
                              asem.at[b]).wait()
        pltpu.async_copy(bufs.at[b], out_hbm.at[pl.ds(base0 + j * CG, CG), :],
                         wsem.at[b])

    def wr_wait(j, b):
        pltpu.make_async_copy(bufs.at[b],
                              out_hbm.at[pl.ds(base0 + j * CG, CG), :],
                              wsem.at[b]).wait()

    def outer(i, carry):
        for b in range(NBUF):
            j = i * NBUF + b

            @pl.when(j >= 2)
            def _():
                wr(j - 2, (b - 2) % NBUF)

            @pl.when(j >= 1)
            def _():
                ga(j - 1, (b - 1) % NBUF)

            @pl.when(j >= NBUF)
            def _():
                wr_wait(j - NBUF, b)

            gs(j, b)
        return carry

    lax.fori_loop(0, NCH // NBUF, outer, 0)
    ga(NCH - 1, (NCH - 1) % NBUF)
    wr(NCH - 2, (NCH - 2) % NBUF)
    wr(NCH - 1, (NCH - 1) % NBUF)
    for b in range(NBUF):
        wr_wait(NCH - NBUF + b, b)


@functools.partial(
    pl.kernel,
    out_type=jax.ShapeDtypeStruct((NC * NP, D), jnp.float32),
    mesh=_mesh,
    scratch_types=[
        pltpu.VMEM((NBUF, CG), jnp.int32),
        pltpu.VMEM((NBUF, CG, D), jnp.float32),
        pltpu.VMEM_SHARED((NP, D), jnp.float32),
        pltpu.SemaphoreType.DMA((NBUF,)),
        pltpu.SemaphoreType.DMA((NBUF,)),
        pltpu.SemaphoreType.DMA((NBUF,)),
    ],
)
def _seg_sum(edge_hbm, rcv_hbm, zeros_hbm, out_hbm, idx, bufs, acc,
             isem, lsem, ssem):
    c = lax.axis_index("c")
    s = lax.axis_index("s")
    w = s * NC + c
    base0 = w * EWH
    pltpu.sync_copy(zeros_hbm, acc.at[pl.ds(s * STRIPE, STRIPE), :])
    plsc.subcore_barrier()

    def ld(j, b):
        pltpu.async_copy(rcv_hbm.at[pl.ds(base0 + j * CG, CG)], idx.at[b],
                         isem.at[b])
        pltpu.async_copy(edge_hbm.at[pl.ds(base0 + j * CG, CG), :], bufs.at[b],
                         lsem.at[b])

    def sadd(j, b):
        pltpu.make_async_copy(rcv_hbm.at[pl.ds(base0 + j * CG, CG)],
                              idx.at[b], isem.at[b]).wait()
        pltpu.make_async_copy(edge_hbm.at[pl.ds(base0 + j * CG, CG), :],
                              bufs.at[b], lsem.at[b]).wait()
        pltpu.async_copy(bufs.at[b], acc.at[idx.at[b]], ssem.at[b], add=True)

    def sadd_wait(b):
        pltpu.make_async_copy(bufs.at[b], acc.at[idx.at[b]],
                              ssem.at[b]).wait()

    def outer(i, carry):
        for b in range(NBUF):
            j = i * NBUF + b

            @pl.when(j >= 1)
            def _():
                sadd(j - 1, (b - 1) % NBUF)

            @pl.when(j >= NBUF)
            def _():
                sadd_wait(b)

            ld(j, b)
        return carry

    lax.fori_loop(0, NCH // NBUF, outer, 0)
    sadd(NCH - 1, (NCH - 1) % NBUF)
    for b in range(NBUF):
        sadd_wait(b)
    plsc.subcore_barrier()
    pltpu.sync_copy(
        acc.at[pl.ds(s * STRIPE, STRIPE), :],
        out_hbm.at[pl.ds(c * NP + s * STRIPE, STRIPE), :],
    )


BN = 1000
BEDGE = 1000
NBE = EH // BEDGE


def _proj_body(nf, w1s, w1r, ps, pr):
    x = nf[...]
    ps[...] = jnp.dot(x, w1s[...], preferred_element_type=jnp.float32)
    pr[...] = jnp.dot(x, w1r[...], preferred_element_type=jnp.float32)


_proj = pl.pallas_call(
    _proj_body,
    grid=(N // BN,),
    in_specs=[
        pl.BlockSpec((BN, D), lambda i: (i, 0)),
        pl.BlockSpec((D, D), lambda i: (0, 0)),
        pl.BlockSpec((D, D), lambda i: (0, 0)),
    ],
    out_specs=[pl.BlockSpec((BN, D), lambda i: (i, 0))] * 2,
    out_shape=[jax.ShapeDtypeStruct((N, D), jnp.float32)] * 2,
)


def _layer_norm(h, g, beta):
    mu = jnp.mean(h, axis=-1, keepdims=True)
    d = h - mu
    var = jnp.mean(d * d, axis=-1, keepdims=True)
    return d * lax.rsqrt(var + 1e-5) * g + beta


def _edge_mlp_body0(gath, ef, w1e, b1, w2, b2, g, beta, new_edge, out_edge):
    e = ef[...]
    h = gath[...] + jnp.dot(e, w1e[...], preferred_element_type=jnp.float32)
    h = jnp.maximum(h + b1[...], 0.0)
    h = jnp.dot(h, w2[...], preferred_element_type=jnp.float32) + b2[...]
    y = _layer_norm(h, g[...], beta[...])
    new_edge[...] = y
    out_edge[...] = y + e


def _edge_mlp_body1(gath, ef, w1e, b1, w2, b2, g, beta, oe_prev,
                    new_edge, out_edge):
    del oe_prev
    _edge_mlp_body0(gath, ef, w1e, b1, w2, b2, g, beta, new_edge, out_edge)


_W_SPECS = [
    pl.BlockSpec((D, D), lambda i: (0, 0)),
    pl.BlockSpec((1, D), lambda i: (0, 0)),
    pl.BlockSpec((D, D), lambda i: (0, 0)),
    pl.BlockSpec((1, D), lambda i: (0, 0)),
    pl.BlockSpec((1, D), lambda i: (0, 0)),
    pl.BlockSpec((1, D), lambda i: (0, 0)),
]

_edge_mlp0 = pl.pallas_call(
    _edge_mlp_body0,
    grid=(NBE,),
    in_specs=[
        pl.BlockSpec((BEDGE, D), lambda i: (i, 0)),
        pl.BlockSpec((BEDGE, D), lambda i: (i, 0)),
    ] + _W_SPECS,
    out_specs=[
        pl.BlockSpec((BEDGE, D), lambda i: (i, 0)),
        pl.BlockSpec((BEDGE, D), lambda i: (i, 0)),
    ],
    out_shape=[
        jax.ShapeDtypeStruct((EH, D), jnp.float32),
        jax.ShapeDtypeStruct((E, D), jnp.float32),
    ],
)

_edge_mlp1 = pl.pallas_call(
    _edge_mlp_body1,
    grid=(NBE,),
    in_specs=[
        pl.BlockSpec((BEDGE, D), lambda i: (i, 0)),
        pl.BlockSpec((BEDGE, D), lambda i: (i + NBE, 0)),
    ] + _W_SPECS + [
        pl.BlockSpec(memory_space=pl.MemorySpace.ANY),
    ],
    out_specs=[
        pl.BlockSpec((BEDGE, D), lambda i: (i, 0)),
        pl.BlockSpec((BEDGE, D), lambda i: (i + NBE, 0)),
    ],
    out_shape=[
        jax.ShapeDtypeStruct((EH, D), jnp.float32),
        jax.ShapeDtypeStruct((E, D), jnp.float32),
    ],
    input_output_aliases={8: 1},
)


def _node_mlp_body(nf, p00, p01, p10, p11, w1a, w1b, b1, w2, b2, g, beta,
                   out):
    x = nf[...]
    seg = (p00[...] + p01[...]) + (p10[...] + p11[...])
    h = jnp.dot(x, w1a[...], preferred_element_type=jnp.float32)
    h = h + jnp.dot(seg, w1b[...], preferred_element_type=jnp.float32)
    h = jnp.maximum(h + b1[...], 0.0)
    h = jnp.dot(h, w2[...], preferred_element_type=jnp.float32) + b2[...]
    out[...] = _layer_norm(h, g[...], beta[...]) + x


_node_mlp = pl.pallas_call(
    _node_mlp_body,
    grid=(N // BN,),
    in_specs=[pl.BlockSpec((BN, D), lambda i: (i, 0))] * 5 + [
        pl.BlockSpec((D, D), lambda i: (0, 0)),
        pl.BlockSpec((D, D), lambda i: (0, 0)),
        pl.BlockSpec((1, D), lambda i: (0, 0)),
        pl.BlockSpec((D, D), lambda i: (0, 0)),
        pl.BlockSpec((1, D), lambda i: (0, 0)),
        pl.BlockSpec((1, D), lambda i: (0, 0)),
        pl.BlockSpec((1, D), lambda i: (0, 0)),
    ],
    out_specs=pl.BlockSpec((BN, D), lambda i: (i, 0)),
    out_shape=jax.ShapeDtypeStruct((N, D), jnp.float32),
)


def kernel(node_features, edge_features, senders, receivers,
           edge_w1, edge_b1, edge_w2, edge_b2, edge_g, edge_beta,
           node_w1, node_b1, node_w2, node_b2, node_g, node_beta):
    ps, pr = _proj(node_features, edge_w1[:D], edge_w1[D:2 * D])
    snd3 = senders.reshape(NHALF, NW, NCH, CG)
    rcv3 = receivers.reshape(NHALF, NW, NCH, CG)
    rcv_h = receivers.reshape(NHALF, EH)
    w1e = edge_w1[2 * D:]
    eb1 = edge_b1.reshape(1, D)
    eb2 = edge_b2.reshape(1, D)
    eg = edge_g.reshape(1, D)
    ebt = edge_beta.reshape(1, D)
    zeros = jnp.zeros((STRIPE, D), jnp.float32)

    g0 = _gather_sum(ps, pr, snd3[0], rcv3[0])
    ne0, oe0 = _edge_mlp0(g0, edge_features, w1e, eb1, edge_w2, eb2, eg, ebt)
    g1 = _gather_sum(ps, pr, snd3[1], rcv3[1])
    parts0 = _seg_sum(ne0, rcv_h[0], zeros)
    ne1, out_edge = _edge_mlp1(g1, edge_features, w1e, eb1, edge_w2, eb2,
                               eg, ebt, oe0)
    parts1 = _seg_sum(ne1, rcv_h[1], zeros)

    out_node = _node_mlp(
        node_features, parts0[:N], parts0[NP:NP + N], parts1[:N],
        parts1[NP:NP + N], node_w1[:D], node_w1[D:],
        node_b1.reshape(1, D), node_w2, node_b2.reshape(1, D),
        node_g.reshape(1, D), node_beta.reshape(1, D))
    return (out_node, out_edge)
